# baseline (speedup 1.0000x reference)
import jax
import jax.numpy as jnp
from jax import lax
from jax.experimental import pallas as pl
from jax.experimental.pallas import tpu as pltpu
from jax.experimental.pallas import tpu_sc as plsc

_N = 10000
_E = 320000
_NPAD = 10240
_DIN = 128
_DH = 256
_G = 64
_EPS = 1e-5

_NT = 16
_CHUNK = 80
_EPT = _E // _NT
_BLK = 4000
_NBLK = _EPT // _BLK
_PAIRS = _BLK // (2 * _CHUNK)
_EPT_DEG = _E // (2 * _NT)
_DBLK = 2000
_NDBLK = _EPT_DEG // _DBLK
_NHALF = _NPAD // 2
_RPP = _NHALF // _NT
_R = 1024


def _deg_body(dst_hbm, out_hbm, dblk, degl, idxv, outst, out_sh):
    c = lax.axis_index("c")
    sid = lax.axis_index("s")

    z = jnp.zeros((16,), jnp.float32)

    def zrow(i, _):
        for k in range(8):
            degl[i, pl.ds(k * 16, 16)] = z
        return 0

    lax.fori_loop(0, 80, zrow, 0)

    @pl.when(sid == 0)
    def _():
        pltpu.sync_copy(degl, out_sh)

    for k in range(5):
        idxv[pl.ds(k * 16, 16)] = lax.iota(jnp.int32, 16) + (k * 16)

    ones = jnp.ones((16,), jnp.float32)

    def blk(b, _):
        base = (c * _NT + sid) * _EPT_DEG + b * _DBLK
        pltpu.sync_copy(dst_hbm.at[pl.ds(base, _DBLK)], dblk)

        def scat(i, _):
            d = dblk[pl.ds(i * 16, 16)]
            plsc.addupdate_scatter(degl, [d >> 7, d & 127], ones)
            return 0

        lax.fori_loop(0, _DBLK // 16, scat, 0)
        return 0

    lax.fori_loop(0, _NDBLK, blk, 0)
    plsc.subcore_barrier()
    pltpu.sync_copy(degl, out_sh.at[idxv], add=True)
    plsc.subcore_barrier()

    @pl.when(sid < 10)
    def _():
        pltpu.sync_copy(out_sh.at[pl.ds(sid * 8, 8)], outst)
        pltpu.sync_copy(outst, out_hbm.at[pl.ds(c * 80 + sid * 8, 8)])


_deg_call = pl.kernel(
    _deg_body,
    out_type=jax.ShapeDtypeStruct((2 * 80, 128), jnp.float32),
    mesh=plsc.VectorSubcoreMesh(core_axis_name="c", subcore_axis_name="s"),
    scratch_types=[
        pltpu.VMEM((_DBLK,), jnp.int32),
        pltpu.VMEM((80, 128), jnp.float32),
        pltpu.VMEM((80,), jnp.int32),
        pltpu.VMEM((8, 128), jnp.float32),
        pltpu.VMEM_SHARED((80, 128), jnp.float32),
    ],
    compiler_params=pltpu.CompilerParams(needs_layout_passes=False),
    name="gnn_deg",
)


def _msg_body(hs2d_hbm, src_hbm, dst_hbm, out_hbm, sblk, dblk, cbs, cbd,
              dbuf0, dbuf1, rows0, rows1, acc, semg0, semg1, sems0, sems1):
    c = lax.axis_index("c")
    sid = lax.axis_index("s")

    z = jnp.zeros((16,), jnp.float32)

    def fire_gather(off, rows, sem):
        pltpu.async_copy(hs2d_hbm.at[cbs.at[pl.ds(off, _CHUNK)]], rows, sem)

    def wait_gather(rows, sem):
        pltpu.make_async_copy(hs2d_hbm.at[pl.ds(0, _CHUNK)], rows, sem).wait()

    def fire_scatter(rows, dbuf, sem):
        pltpu.async_copy(rows, acc.at[dbuf], sem, add=True)

    def wait_scatter(rows, sem):
        pltpu.make_async_copy(rows, acc.at[pl.ds(0, _CHUNK)], sem).wait()

    trash = jnp.full((16,), _NHALF, jnp.int32)
    zi = jnp.zeros((16,), jnp.int32)

    for p in range(2):
        def copy_dbuf(dbuf, off):
            for k in range(_CHUNK // 16):
                dbuf[pl.ds(k * 16, 16)] = cbd[pl.ds(off + k * 16, 16)]

        def zrow(i, _):
            for k in range(8):
                rows0[i, pl.ds(k * 16, 16)] = z
            return 0

        lax.fori_loop(0, _CHUNK, zrow, 0)
        for q in range(_RPP // _CHUNK):
            pltpu.sync_copy(rows0, acc.at[pl.ds(sid * _RPP + q * _CHUNK, _CHUNK)])
        plsc.subcore_barrier()

        for b in range(_NBLK):
            base = sid * _EPT + b * _BLK
            pltpu.sync_copy(src_hbm.at[pl.ds(base, _BLK)], sblk)
            pltpu.sync_copy(dst_hbm.at[pl.ds(base, _BLK)], dblk)

            def pre(i, _):
                cbd[pl.ds(i * 16, 16)] = trash
                cbs[pl.ds(i * 16, 16)] = zi
                return 0

            lax.fori_loop(0, _BLK // 16, pre, 0)

            def compact(i, off):
                d = dblk[pl.ds(i * 16, 16)] - (p * _NHALF)
                ok = (d >= 0) & (d < _NHALF)
                s2 = sblk[pl.ds(i * 16, 16)] * 2 + c
                plsc.store_compressed(cbd.at[pl.ds(off, 16)], d, mask=ok)
                plsc.store_compressed(cbs.at[pl.ds(off, 16)], s2, mask=ok)
                return off + jnp.sum(ok.astype(jnp.int32))

            cnt = lax.fori_loop(0, _BLK // 16, compact, jnp.int32(0))
            npairs = lax.div(cnt + (2 * _CHUNK - 1), jnp.int32(2 * _CHUNK))

            for k in range(_CHUNK // 16):
                dbuf1[pl.ds(k * 16, 16)] = trash
            fire_scatter(rows1, dbuf1, sems1)

            @pl.when(npairs > 0)
            def _():
                copy_dbuf(dbuf0, 0)
                fire_gather(0, rows0, semg0)

            def pair(i, _):
                ea = i * 2 * _CHUNK
                eb = ea + _CHUNK
                en = eb + _CHUNK
                wait_scatter(rows1, sems1)
                wait_gather(rows0, semg0)
                fire_scatter(rows0, dbuf0, sems0)
                copy_dbuf(dbuf1, eb)
                fire_gather(eb, rows1, semg1)
                wait_scatter(rows0, sems0)

                @pl.when(i < npairs - 1)
                def _():
                    copy_dbuf(dbuf0, en)
                    fire_gather(en, rows0, semg0)

                wait_gather(rows1, semg1)
                fire_scatter(rows1, dbuf1, sems1)
                return 0

            lax.fori_loop(0, npairs, pair, 0)
            wait_scatter(rows1, sems1)

        plsc.subcore_barrier()

        for q in range(_RPP // _CHUNK):
            r = sid * _RPP + q * _CHUNK
            pltpu.sync_copy(acc.at[pl.ds(r, _CHUNK)], rows0)
            pltpu.sync_copy(rows0, out_hbm.at[pl.ds(c * _NPAD + p * _NHALF + r, _CHUNK)])
        if p == 0:
            plsc.subcore_barrier()


_msg_call = pl.kernel(
    _msg_body,
    out_type=jax.ShapeDtypeStruct((2 * _NPAD, 128), jnp.float32),
    mesh=plsc.VectorSubcoreMesh(core_axis_name="c", subcore_axis_name="s"),
    scratch_types=[
        pltpu.VMEM((_BLK,), jnp.int32),
        pltpu.VMEM((_BLK,), jnp.int32),
        pltpu.VMEM((_BLK + 96,), jnp.int32),
        pltpu.VMEM((_BLK + 96,), jnp.int32),
        pltpu.VMEM((_CHUNK,), jnp.int32),
        pltpu.VMEM((_CHUNK,), jnp.int32),
        pltpu.VMEM((_CHUNK, 128), jnp.float32),
        pltpu.VMEM((_CHUNK, 128), jnp.float32),
        pltpu.VMEM_SHARED((_NHALF + 8, 128), jnp.float32),
        pltpu.SemaphoreType.DMA,
        pltpu.SemaphoreType.DMA,
        pltpu.SemaphoreType.DMA,
        pltpu.SemaphoreType.DMA,
    ],
    compiler_params=pltpu.CompilerParams(needs_layout_passes=False),
    name="gnn_msg",
)


def _dis_block(dga_ref, dgb_ref):
    deg = dga_ref[...] + dgb_ref[...] + 1.0
    return lax.rsqrt(deg)


def _bn_rs():
    return lax.rsqrt(jnp.asarray(1.0 + _EPS, jnp.float32))


def _tcb_body(x_ref, dga_ref, dgb_ref, g0_ref, b0_ref, pw_ref, pb_ref, w1_ref, out_ref):
    x = x_ref[...]
    h = (g0_ref[...] * _bn_rs()) * x + b0_ref[...]
    h = jnp.maximum(jnp.dot(h, pw_ref[...], preferred_element_type=jnp.float32)
                    + pb_ref[...], 0.0)
    hw = jnp.dot(h, w1_ref[...], preferred_element_type=jnp.float32)
    out_ref[...] = hw * _dis_block(dga_ref, dgb_ref)


def _tcb(xp, dga, dgb, g0, b0, pw, pb, w1):
    grid = _NPAD // _R
    return pl.pallas_call(
        _tcb_body,
        grid=(grid,),
        in_specs=[
            pl.BlockSpec((_R, _DIN), lambda i: (i, 0)),
            pl.BlockSpec((_R, 1), lambda i: (i, 0)),
            pl.BlockSpec((_R, 1), lambda i: (i, 0)),
            pl.BlockSpec((1, _DIN), lambda i: (0, 0)),
            pl.BlockSpec((1, _DIN), lambda i: (0, 0)),
            pl.BlockSpec((_DIN, _DH), lambda i: (0, 0)),
            pl.BlockSpec((1, _DH), lambda i: (0, 0)),
            pl.BlockSpec((_DH, _DH), lambda i: (0, 0)),
        ],
        out_specs=pl.BlockSpec((_R, _DH), lambda i: (i, 0)),
        out_shape=jax.ShapeDtypeStruct((_NPAD, _DH), jnp.float32),
    )(xp, dga, dgb, g0, b0, pw, pb, w1)


def _tcc_body(a0_ref, a1_ref, hs_ref, dga_ref, dgb_ref, b1_ref, g1_ref, bb1_ref,
              w2_ref, out_ref):
    dis = _dis_block(dga_ref, dgb_ref)
    a = jnp.concatenate([a0_ref[...], a1_ref[...]], axis=1)
    o = (a + hs_ref[...]) * dis + b1_ref[...]
    h = jnp.maximum((g1_ref[...] * _bn_rs()) * o + bb1_ref[...], 0.0)
    hw = jnp.dot(h, w2_ref[...], preferred_element_type=jnp.float32)
    out_ref[...] = hw * dis


def _tcc(a0, a1, hs, dga, dgb, b1, g1, bb1, w2):
    grid = _NPAD // _R
    return pl.pallas_call(
        _tcc_body,
        grid=(grid,),
        in_specs=[
            pl.BlockSpec((_R, 128), lambda i: (i, 0)),
            pl.BlockSpec((_R, 128), lambda i: (i, 0)),
            pl.BlockSpec((_R, _DH), lambda i: (i, 0)),
            pl.BlockSpec((_R, 1), lambda i: (i, 0)),
            pl.BlockSpec((_R, 1), lambda i: (i, 0)),
            pl.BlockSpec((1, _DH), lambda i: (0, 0)),
            pl.BlockSpec((1, _DH), lambda i: (0, 0)),
            pl.BlockSpec((1, _DH), lambda i: (0, 0)),
            pl.BlockSpec((_DH, _DH), lambda i: (0, 0)),
        ],
        out_specs=pl.BlockSpec((_R, _DH), lambda i: (i, 0)),
        out_shape=jax.ShapeDtypeStruct((_NPAD, _DH), jnp.float32),
    )(a0, a1, hs, dga, dgb, b1, g1, bb1, w2)


def _tcd_body(a0_ref, a1_ref, hs_ref, dga_ref, dgb_ref, b2_ref, g2_ref, bb2_ref,
              bat_ref, c1w_ref, c1b_ref, c2w_ref, c2b_ref, out_ref,
              sums_ref, cnt_ref):
    i = pl.program_id(0)
    dis = _dis_block(dga_ref, dgb_ref)
    a = jnp.concatenate([a0_ref[...], a1_ref[...]], axis=1)
    o = (a + hs_ref[...]) * dis + b2_ref[...]
    h = jnp.maximum((g2_ref[...] * _bn_rs()) * o + bb2_ref[...], 0.0)

    bids = bat_ref[...]
    gids = lax.broadcasted_iota(jnp.int32, (1, _G), 1)
    mask = (bids == gids).astype(jnp.float32)

    @pl.when(i == 0)
    def _():
        sums_ref[...] = jnp.zeros_like(sums_ref)
        cnt_ref[...] = jnp.zeros_like(cnt_ref)

    dn = (((0,), (0,)), ((), ()))
    sums_ref[...] += lax.dot_general(mask, h, dn, preferred_element_type=jnp.float32)
    cnt_ref[...] += lax.dot_general(mask, jnp.ones((_R, 1), jnp.float32), dn,
                                    preferred_element_type=jnp.float32)

    @pl.when(i == pl.num_programs(0) - 1)
    def _():
        pooled = sums_ref[...] / jnp.maximum(cnt_ref[...], 1.0)
        z = jnp.maximum(jnp.dot(pooled, c1w_ref[...],
                                preferred_element_type=jnp.float32) + c1b_ref[...], 0.0)
        out_ref[...] = jnp.dot(z, c2w_ref[...],
                               preferred_element_type=jnp.float32) + c2b_ref[...]


def _tcd(a0, a1, hs, dga, dgb, b2, g2, bb2, batp, c1w, c1b, c2w, c2b):
    grid = _NPAD // _R
    return pl.pallas_call(
        _tcd_body,
        grid=(grid,),
        in_specs=[
            pl.BlockSpec((_R, 128), lambda i: (i, 0)),
            pl.BlockSpec((_R, 128), lambda i: (i, 0)),
            pl.BlockSpec((_R, _DH), lambda i: (i, 0)),
            pl.BlockSpec((_R, 1), lambda i: (i, 0)),
            pl.BlockSpec((_R, 1), lambda i: (i, 0)),
            pl.BlockSpec((1, _DH), lambda i: (0, 0)),
            pl.BlockSpec((1, _DH), lambda i: (0, 0)),
            pl.BlockSpec((1, _DH), lambda i: (0, 0)),
            pl.BlockSpec((_R, 1), lambda i: (i, 0)),
            pl.BlockSpec((_DH, _DH // 2), lambda i: (0, 0)),
            pl.BlockSpec((1, _DH // 2), lambda i: (0, 0)),
            pl.BlockSpec((_DH // 2, 128), lambda i: (0, 0)),
            pl.BlockSpec((1, 128), lambda i: (0, 0)),
        ],
        out_specs=pl.BlockSpec((_G, 128), lambda i: (0, 0)),
        out_shape=jax.ShapeDtypeStruct((_G, 128), jnp.float32),
        scratch_shapes=[
            pltpu.VMEM((_G, _DH), jnp.float32),
            pltpu.VMEM((_G, 1), jnp.float32),
        ],
    )(a0, a1, hs, dga, dgb, b2, g2, bb2, batp, c1w, c1b, c2w, c2b)


def kernel(x, edge_index, batch, bn0_g, bn0_b, proj_W, proj_b, conv1_W, conv1_b,
           bn1_g, bn1_b, conv2_W, conv2_b, bn2_g, bn2_b, clf1_W, clf1_b,
           clf2_W, clf2_b):
    src = edge_index[0]
    dst = edge_index[1]

    deg2d = _deg_call(dst)
    degflat = deg2d.reshape(2, _NPAD)
    dga = degflat[0].reshape(_NPAD, 1)
    dgb = degflat[1].reshape(_NPAD, 1)

    xp = jnp.pad(x, ((0, _NPAD - _N), (0, 0)))
    hs1 = _tcb(xp, dga, dgb, bn0_g.reshape(1, -1), bn0_b.reshape(1, -1),
               proj_W, proj_b.reshape(1, -1), conv1_W)

    agg1 = _msg_call(hs1.reshape(2 * _NPAD, 128), src, dst)
    hs2 = _tcc(agg1[:_NPAD], agg1[_NPAD:], hs1, dga, dgb,
               conv1_b.reshape(1, -1), bn1_g.reshape(1, -1), bn1_b.reshape(1, -1),
               conv2_W)

    agg2 = _msg_call(hs2.reshape(2 * _NPAD, 128), src, dst)

    batp = jnp.pad(batch, (0, _NPAD - _N), constant_values=_G).reshape(_NPAD, 1)
    c2w = jnp.pad(clf2_W, ((0, 0), (0, 126)))
    c2b = jnp.pad(clf2_b, (0, 126)).reshape(1, -1)
    outp = _tcd(agg2[:_NPAD], agg2[_NPAD:], hs2, dga, dgb,
                conv2_b.reshape(1, -1), bn2_g.reshape(1, -1), bn2_b.reshape(1, -1),
                batp, clf1_W, clf1_b.reshape(1, -1), c2w, c2b)
    return outp[:, :2]


# speedup vs baseline: 2.6235x; 2.6235x over previous
import jax
import jax.numpy as jnp
from jax import lax
from jax.experimental import pallas as pl
from jax.experimental.pallas import tpu as pltpu
from jax.experimental.pallas import tpu_sc as plsc

_N = 10000
_E = 320000
_NPAD = 10240
_DIN = 128
_DH = 256
_G = 64
_EPS = 1e-5

_NT = 16
_CHUNK = 80
_EPT = _E // _NT
_BLK = 4000
_NBLK = _EPT // _BLK
_PAIRS = _BLK // (2 * _CHUNK)
_EPT_DEG = _E // (2 * _NT)
_DBLK = 2000
_NDBLK = _EPT_DEG // _DBLK
_RPT = _NPAD // _NT
_R = 1024


def _deg_body(dst_hbm, out_hbm, dblk, degl, idxv, outst, out_sh):
    c = lax.axis_index("c")
    sid = lax.axis_index("s")

    z = jnp.zeros((16,), jnp.float32)

    def zrow(i, _):
        for k in range(8):
            degl[i, pl.ds(k * 16, 16)] = z
        return 0

    lax.fori_loop(0, 80, zrow, 0)

    @pl.when(sid == 0)
    def _():
        pltpu.sync_copy(degl, out_sh)

    for k in range(5):
        idxv[pl.ds(k * 16, 16)] = lax.iota(jnp.int32, 16) + (k * 16)

    ones = jnp.ones((16,), jnp.float32)

    def blk(b, _):
        base = (c * _NT + sid) * _EPT_DEG + b * _DBLK
        pltpu.sync_copy(dst_hbm.at[pl.ds(base, _DBLK)], dblk)

        def scat(i, _):
            d = dblk[pl.ds(i * 16, 16)]
            plsc.addupdate_scatter(degl, [d >> 7, d & 127], ones)
            return 0

        lax.fori_loop(0, _DBLK // 16, scat, 0)
        return 0

    lax.fori_loop(0, _NDBLK, blk, 0)
    plsc.subcore_barrier()
    pltpu.sync_copy(degl, out_sh.at[idxv], add=True)
    plsc.subcore_barrier()

    @pl.when(sid < 10)
    def _():
        pltpu.sync_copy(out_sh.at[pl.ds(sid * 8, 8)], outst)
        pltpu.sync_copy(outst, out_hbm.at[pl.ds(c * 80 + sid * 8, 8)])


_deg_call = pl.kernel(
    _deg_body,
    out_type=jax.ShapeDtypeStruct((2 * 80, 128), jnp.float32),
    mesh=plsc.VectorSubcoreMesh(core_axis_name="c", subcore_axis_name="s"),
    scratch_types=[
        pltpu.VMEM((_DBLK,), jnp.int32),
        pltpu.VMEM((80, 128), jnp.float32),
        pltpu.VMEM((80,), jnp.int32),
        pltpu.VMEM((8, 128), jnp.float32),
        pltpu.VMEM_SHARED((80, 128), jnp.float32),
    ],
    compiler_params=pltpu.CompilerParams(needs_layout_passes=False),
    name="gnn_deg",
)


def _msg_body(hs4_hbm, src_hbm, dst_hbm, out_hbm, sblk, dblk, dbuf0, dbuf1,
              rows0, rows1, acc, semg0, semg1, sems0, sems1):
    c = lax.axis_index("c")
    sid = lax.axis_index("s")

    z = jnp.zeros((16,), jnp.float32)

    def fire_gather(off, rows, sem):
        pltpu.async_copy(hs4_hbm.at[sblk.at[pl.ds(off, _CHUNK)]], rows, sem)

    def wait_gather(rows, sem):
        pltpu.make_async_copy(hs4_hbm.at[pl.ds(0, _CHUNK)], rows, sem).wait()

    def fire_scatter(rows, dbuf, sem):
        pltpu.async_copy(rows, acc.at[dbuf], sem, add=True)

    def wait_scatter(rows, sem):
        pltpu.make_async_copy(rows, acc.at[pl.ds(0, _CHUNK)], sem).wait()

    def fill(dbuf, off):
        for k in range(_CHUNK // 16):
            dbuf[pl.ds(k * 16, 16)] = dblk[pl.ds(off + k * 16, 16)]

    for p in range(2):
        def zrow(i, _):
            for k in range(4):
                rows0[i, pl.ds(k * 16, 16)] = z
            return 0

        lax.fori_loop(0, _CHUNK, zrow, 0)
        for q in range(_RPT // _CHUNK):
            pltpu.sync_copy(rows0, acc.at[pl.ds(sid * _RPT + q * _CHUNK, _CHUNK)])
        plsc.subcore_barrier()

        for b in range(_NBLK):
            base = sid * _EPT + b * _BLK
            pltpu.sync_copy(src_hbm.at[pl.ds(base, _BLK)], sblk)
            pltpu.sync_copy(dst_hbm.at[pl.ds(base, _BLK)], dblk)

            def adj(i, _):
                s = sblk[pl.ds(i * 16, 16)]
                sblk[pl.ds(i * 16, 16)] = s * 4 + (c * 2 + p)
                return 0

            lax.fori_loop(0, _BLK // 16, adj, 0)

            trash = jnp.full((16,), _NPAD, jnp.int32)
            for k in range(_CHUNK // 16):
                dbuf1[pl.ds(k * 16, 16)] = trash
            fire_scatter(rows1, dbuf1, sems1)
            fill(dbuf0, 0)
            fire_gather(0, rows0, semg0)

            def pair(i, _):
                ea = i * 2 * _CHUNK
                eb = ea + _CHUNK
                en = eb + _CHUNK
                wait_scatter(rows1, sems1)
                wait_gather(rows0, semg0)
                fire_scatter(rows0, dbuf0, sems0)
                fill(dbuf1, eb)
                fire_gather(eb, rows1, semg1)
                wait_scatter(rows0, sems0)

                @pl.when(i < _PAIRS - 1)
                def _():
                    fill(dbuf0, en)
                    fire_gather(en, rows0, semg0)

                wait_gather(rows1, semg1)
                fire_scatter(rows1, dbuf1, sems1)
                return 0

            lax.fori_loop(0, _PAIRS, pair, 0)
            wait_scatter(rows1, sems1)

        plsc.subcore_barrier()

        for q in range(_RPT // _CHUNK):
            r = sid * _RPT + q * _CHUNK
            pltpu.sync_copy(acc.at[pl.ds(r, _CHUNK)], rows0)
            pltpu.sync_copy(rows0, out_hbm.at[pl.ds((c * 2 + p) * _NPAD + r, _CHUNK)])
        if p == 0:
            plsc.subcore_barrier()


_msg_call = pl.kernel(
    _msg_body,
    out_type=jax.ShapeDtypeStruct((4 * _NPAD, 64), jnp.float32),
    mesh=plsc.VectorSubcoreMesh(core_axis_name="c", subcore_axis_name="s"),
    scratch_types=[
        pltpu.VMEM((_BLK,), jnp.int32),
        pltpu.VMEM((_BLK,), jnp.int32),
        pltpu.VMEM((_CHUNK,), jnp.int32),
        pltpu.VMEM((_CHUNK,), jnp.int32),
        pltpu.VMEM((_CHUNK, 64), jnp.float32),
        pltpu.VMEM((_CHUNK, 64), jnp.float32),
        pltpu.VMEM_SHARED((_NPAD + 8, 64), jnp.float32),
        pltpu.SemaphoreType.DMA,
        pltpu.SemaphoreType.DMA,
        pltpu.SemaphoreType.DMA,
        pltpu.SemaphoreType.DMA,
    ],
    compiler_params=pltpu.CompilerParams(needs_layout_passes=False,
                                         use_tc_tiling_on_sc=False),
    name="gnn_msg",
)


def _dis_block(dga_ref, dgb_ref):
    deg = dga_ref[...] + dgb_ref[...] + 1.0
    return lax.rsqrt(deg)


def _bn_rs():
    return lax.rsqrt(jnp.asarray(1.0 + _EPS, jnp.float32))


def _tcb_body(x_ref, dga_ref, dgb_ref, g0_ref, b0_ref, pw_ref, pb_ref, w1_ref, out_ref):
    x = x_ref[...]
    h = (g0_ref[...] * _bn_rs()) * x + b0_ref[...]
    h = jnp.maximum(jnp.dot(h, pw_ref[...], preferred_element_type=jnp.float32)
                    + pb_ref[...], 0.0)
    hw = jnp.dot(h, w1_ref[...], preferred_element_type=jnp.float32)
    out_ref[...] = hw * _dis_block(dga_ref, dgb_ref)


def _tcb(xp, dga, dgb, g0, b0, pw, pb, w1):
    grid = _NPAD // _R
    return pl.pallas_call(
        _tcb_body,
        grid=(grid,),
        in_specs=[
            pl.BlockSpec((_R, _DIN), lambda i: (i, 0)),
            pl.BlockSpec((_R, 1), lambda i: (i, 0)),
            pl.BlockSpec((_R, 1), lambda i: (i, 0)),
            pl.BlockSpec((1, _DIN), lambda i: (0, 0)),
            pl.BlockSpec((1, _DIN), lambda i: (0, 0)),
            pl.BlockSpec((_DIN, _DH), lambda i: (0, 0)),
            pl.BlockSpec((1, _DH), lambda i: (0, 0)),
            pl.BlockSpec((_DH, _DH), lambda i: (0, 0)),
        ],
        out_specs=pl.BlockSpec((_R, _DH), lambda i: (i, 0)),
        out_shape=jax.ShapeDtypeStruct((_NPAD, _DH), jnp.float32),
    )(xp, dga, dgb, g0, b0, pw, pb, w1)


def _tcc_body(a0_ref, a1_ref, a2_ref, a3_ref, hs_ref, dga_ref, dgb_ref,
              b1_ref, g1_ref, bb1_ref, w2_ref, out_ref):
    dis = _dis_block(dga_ref, dgb_ref)
    a = jnp.concatenate([a0_ref[...], a1_ref[...], a2_ref[...], a3_ref[...]],
                        axis=1)
    o = (a + hs_ref[...]) * dis + b1_ref[...]
    h = jnp.maximum((g1_ref[...] * _bn_rs()) * o + bb1_ref[...], 0.0)
    hw = jnp.dot(h, w2_ref[...], preferred_element_type=jnp.float32)
    out_ref[...] = hw * dis


def _tcc(a0, a1, a2, a3, hs, dga, dgb, b1, g1, bb1, w2):
    grid = _NPAD // _R
    return pl.pallas_call(
        _tcc_body,
        grid=(grid,),
        in_specs=[
            pl.BlockSpec((_R, 64), lambda i: (i, 0)),
            pl.BlockSpec((_R, 64), lambda i: (i, 0)),
            pl.BlockSpec((_R, 64), lambda i: (i, 0)),
            pl.BlockSpec((_R, 64), lambda i: (i, 0)),
            pl.BlockSpec((_R, _DH), lambda i: (i, 0)),
            pl.BlockSpec((_R, 1), lambda i: (i, 0)),
            pl.BlockSpec((_R, 1), lambda i: (i, 0)),
            pl.BlockSpec((1, _DH), lambda i: (0, 0)),
            pl.BlockSpec((1, _DH), lambda i: (0, 0)),
            pl.BlockSpec((1, _DH), lambda i: (0, 0)),
            pl.BlockSpec((_DH, _DH), lambda i: (0, 0)),
        ],
        out_specs=pl.BlockSpec((_R, _DH), lambda i: (i, 0)),
        out_shape=jax.ShapeDtypeStruct((_NPAD, _DH), jnp.float32),
    )(a0, a1, a2, a3, hs, dga, dgb, b1, g1, bb1, w2)


def _tcd_body(a0_ref, a1_ref, a2_ref, a3_ref, hs_ref, dga_ref, dgb_ref,
              b2_ref, g2_ref, bb2_ref, bat_ref, c1w_ref, c1b_ref, c2w_ref,
              c2b_ref, out_ref, sums_ref, cnt_ref):
    i = pl.program_id(0)
    dis = _dis_block(dga_ref, dgb_ref)
    a = jnp.concatenate([a0_ref[...], a1_ref[...], a2_ref[...], a3_ref[...]],
                        axis=1)
    o = (a + hs_ref[...]) * dis + b2_ref[...]
    h = jnp.maximum((g2_ref[...] * _bn_rs()) * o + bb2_ref[...], 0.0)

    bids = bat_ref[...]
    gids = lax.broadcasted_iota(jnp.int32, (1, _G), 1)
    mask = (bids == gids).astype(jnp.float32)

    @pl.when(i == 0)
    def _():
        sums_ref[...] = jnp.zeros_like(sums_ref)
        cnt_ref[...] = jnp.zeros_like(cnt_ref)

    dn = (((0,), (0,)), ((), ()))
    sums_ref[...] += lax.dot_general(mask, h, dn, preferred_element_type=jnp.float32)
    cnt_ref[...] += lax.dot_general(mask, jnp.ones((_R, 1), jnp.float32), dn,
                                    preferred_element_type=jnp.float32)

    @pl.when(i == pl.num_programs(0) - 1)
    def _():
        pooled = sums_ref[...] / jnp.maximum(cnt_ref[...], 1.0)
        z = jnp.maximum(jnp.dot(pooled, c1w_ref[...],
                                preferred_element_type=jnp.float32) + c1b_ref[...], 0.0)
        out_ref[...] = jnp.dot(z, c2w_ref[...],
                               preferred_element_type=jnp.float32) + c2b_ref[...]


def _tcd(a0, a1, a2, a3, hs, dga, dgb, b2, g2, bb2, batp, c1w, c1b, c2w, c2b):
    grid = _NPAD // _R
    return pl.pallas_call(
        _tcd_body,
        grid=(grid,),
        in_specs=[
            pl.BlockSpec((_R, 64), lambda i: (i, 0)),
            pl.BlockSpec((_R, 64), lambda i: (i, 0)),
            pl.BlockSpec((_R, 64), lambda i: (i, 0)),
            pl.BlockSpec((_R, 64), lambda i: (i, 0)),
            pl.BlockSpec((_R, _DH), lambda i: (i, 0)),
            pl.BlockSpec((_R, 1), lambda i: (i, 0)),
            pl.BlockSpec((_R, 1), lambda i: (i, 0)),
            pl.BlockSpec((1, _DH), lambda i: (0, 0)),
            pl.BlockSpec((1, _DH), lambda i: (0, 0)),
            pl.BlockSpec((1, _DH), lambda i: (0, 0)),
            pl.BlockSpec((_R, 1), lambda i: (i, 0)),
            pl.BlockSpec((_DH, _DH // 2), lambda i: (0, 0)),
            pl.BlockSpec((1, _DH // 2), lambda i: (0, 0)),
            pl.BlockSpec((_DH // 2, 128), lambda i: (0, 0)),
            pl.BlockSpec((1, 128), lambda i: (0, 0)),
        ],
        out_specs=pl.BlockSpec((_G, 128), lambda i: (0, 0)),
        out_shape=jax.ShapeDtypeStruct((_G, 128), jnp.float32),
        scratch_shapes=[
            pltpu.VMEM((_G, _DH), jnp.float32),
            pltpu.VMEM((_G, 1), jnp.float32),
        ],
    )(a0, a1, a2, a3, hs, dga, dgb, b2, g2, bb2, batp, c1w, c1b, c2w, c2b)


def kernel(x, edge_index, batch, bn0_g, bn0_b, proj_W, proj_b, conv1_W, conv1_b,
           bn1_g, bn1_b, conv2_W, conv2_b, bn2_g, bn2_b, clf1_W, clf1_b,
           clf2_W, clf2_b):
    src = edge_index[0]
    dst = edge_index[1]

    deg2d = _deg_call(dst)
    degflat = deg2d.reshape(2, _NPAD)
    dga = degflat[0].reshape(_NPAD, 1)
    dgb = degflat[1].reshape(_NPAD, 1)

    xp = jnp.pad(x, ((0, _NPAD - _N), (0, 0)))
    hs1 = _tcb(xp, dga, dgb, bn0_g.reshape(1, -1), bn0_b.reshape(1, -1),
               proj_W, proj_b.reshape(1, -1), conv1_W)

    agg1 = _msg_call(hs1.reshape(4 * _NPAD, 64), src, dst)
    a1q = agg1.reshape(4, _NPAD, 64)
    hs2 = _tcc(a1q[0], a1q[1], a1q[2], a1q[3], hs1, dga, dgb,
               conv1_b.reshape(1, -1), bn1_g.reshape(1, -1), bn1_b.reshape(1, -1),
               conv2_W)

    agg2 = _msg_call(hs2.reshape(4 * _NPAD, 64), src, dst)
    a2q = agg2.reshape(4, _NPAD, 64)

    batp = jnp.pad(batch, (0, _NPAD - _N), constant_values=_G).reshape(_NPAD, 1)
    c2w = jnp.pad(clf2_W, ((0, 0), (0, 126)))
    c2b = jnp.pad(clf2_b, (0, 126)).reshape(1, -1)
    outp = _tcd(a2q[0], a2q[1], a2q[2], a2q[3], hs2, dga, dgb,
                conv2_b.reshape(1, -1), bn2_g.reshape(1, -1), bn2_b.reshape(1, -1),
                batp, clf1_W, clf1_b.reshape(1, -1), c2w, c2b)
    return outp[:, :2]
